# Initial kernel scaffold; baseline (speedup 1.0000x reference)
import jax
import jax.numpy as jnp
from jax import lax
from jax.experimental import pallas as pl
from jax.experimental.pallas import tpu as pltpu
from jax.experimental.pallas import tpu_sc as plsc

N = 10000
D = 128
E = 320000
NC = 2
NS = 16
NW = NC * NS

CHUNK = 128
E_PAD = 327680
ROWS_PER_W = E_PAD // NW // CHUNK
NPAD = 10016
ROWS_PER_TILE = NPAD // NS
CNT_W = 16


def _relu_body(x_ref, o_ref):
    o_ref[...] = jnp.maximum(x_ref[...], 0.0)


def _relu(x):
    return pl.pallas_call(
        _relu_body,
        out_shape=jax.ShapeDtypeStruct((N, D), jnp.float32),
    )(x)


def _sc_body(h_hbm, src_hbm, dst_hbm, zagg_hbm, zcnt_hbm,
             agg_out, cnt_out,
             src_idx, dst_idx, rows_v, ones_v,
             sh_agg, sh_cnt, sem):
    c = lax.axis_index("c")
    s = lax.axis_index("s")
    wid = c * NS + s

    r0 = s * ROWS_PER_TILE
    pltpu.sync_copy(zagg_hbm.at[pl.ds(r0, ROWS_PER_TILE)],
                    sh_agg.at[pl.ds(r0, ROWS_PER_TILE)])
    pltpu.sync_copy(zcnt_hbm.at[pl.ds(r0, ROWS_PER_TILE)],
                    sh_cnt.at[pl.ds(r0, ROWS_PER_TILE)])

    ones16 = jnp.full((16,), 1.0, dtype=jnp.float32)

    @pl.loop(0, CHUNK)
    def _fill(r):
        ones_v[r, :] = ones16

    plsc.subcore_barrier()

    row0 = wid * ROWS_PER_W

    @pl.loop(0, ROWS_PER_W)
    def _edges(i):
        r = row0 + i
        pltpu.sync_copy(src_hbm.at[r], src_idx)
        pltpu.sync_copy(dst_hbm.at[r], dst_idx)
        pltpu.async_copy(h_hbm.at[src_idx], rows_v, sem).wait()
        pltpu.sync_copy(rows_v, sh_agg.at[dst_idx], add=True)
        pltpu.sync_copy(ones_v, sh_cnt.at[dst_idx], add=True)

    plsc.subcore_barrier()

    pltpu.sync_copy(sh_agg.at[pl.ds(r0, ROWS_PER_TILE)],
                    agg_out.at[c].at[pl.ds(r0, ROWS_PER_TILE)])
    pltpu.sync_copy(sh_cnt.at[pl.ds(r0, ROWS_PER_TILE)],
                    cnt_out.at[c].at[pl.ds(r0, ROWS_PER_TILE)])


def _sc_aggregate(h, src_r, dst_r, zagg, zcnt):
    mesh = plsc.VectorSubcoreMesh(core_axis_name="c", subcore_axis_name="s")
    k = pl.kernel(
        _sc_body,
        out_type=(
            jax.ShapeDtypeStruct((NC, NPAD, D), jnp.float32),
            jax.ShapeDtypeStruct((NC, NPAD, CNT_W), jnp.float32),
        ),
        mesh=mesh,
        scratch_types=[
            pltpu.VMEM((CHUNK,), jnp.int32),
            pltpu.VMEM((CHUNK,), jnp.int32),
            pltpu.VMEM((CHUNK, D), jnp.float32),
            pltpu.VMEM((CHUNK, CNT_W), jnp.float32),
            pltpu.VMEM_SHARED((NPAD, D), jnp.float32),
            pltpu.VMEM_SHARED((NPAD, CNT_W), jnp.float32),
            pltpu.SemaphoreType.DMA,
        ],
    )
    return k(h, src_r, dst_r, zagg, zcnt)


def _combine_body(h_ref, agg_ref, cnt_ref, wl_ref, bl_ref, wr_ref,
                  g_ref, b_ref, o_ref):
    agg = agg_ref[0, :N, :] + agg_ref[1, :N, :]
    cnt = cnt_ref[0, :N, 0:1] + cnt_ref[1, :N, 0:1]
    mean_agg = agg / jnp.maximum(cnt, 1.0)
    h = h_ref[...]
    out = (
        lax.dot_general(mean_agg, wl_ref[...], (((1,), (1,)), ((), ())),
                        preferred_element_type=jnp.float32)
        + bl_ref[...]
        + lax.dot_general(h, wr_ref[...], (((1,), (1,)), ((), ())),
                          preferred_element_type=jnp.float32)
    )
    mu = jnp.mean(out, axis=0, keepdims=True)
    ctr = out - mu
    var = jnp.mean(ctr * ctr, axis=0, keepdims=True)
    o_ref[...] = ctr * (g_ref[...] * lax.rsqrt(var + 1e-5)) + b_ref[...]


def _combine(h, agg, cnt, Wl, bl, Wr, gamma, beta):
    return pl.pallas_call(
        _combine_body,
        out_shape=jax.ShapeDtypeStruct((N, D), jnp.float32),
    )(h, agg, cnt, Wl, bl.reshape(1, D), Wr,
      gamma.reshape(1, D), beta.reshape(1, D))


def kernel(x, edge_index, edge_attr, Wl, bl, Wr, gamma, beta):
    del edge_attr
    h = _relu(x)

    pad = E_PAD - E
    src = jnp.concatenate([edge_index[0], jnp.zeros((pad,), jnp.int32)])
    dst = jnp.concatenate([edge_index[1], jnp.full((pad,), N, jnp.int32)])
    src_r = src.reshape(E_PAD // CHUNK, CHUNK)
    dst_r = dst.reshape(E_PAD // CHUNK, CHUNK)
    zagg = jnp.zeros((NPAD, D), jnp.float32)
    zcnt = jnp.zeros((NPAD, CNT_W), jnp.float32)

    agg, cnt = _sc_aggregate(h, src_r, dst_r, zagg, zcnt)
    return _combine(h, agg, cnt, Wl, bl, Wr, gamma, beta)


# speedup vs baseline: 1.9182x; 1.9182x over previous
import jax
import jax.numpy as jnp
from jax import lax
from jax.experimental import pallas as pl
from jax.experimental.pallas import tpu as pltpu
from jax.experimental.pallas import tpu_sc as plsc

N = 10000
D = 128
E = 320000
NC = 2
NS = 16

CHUNK = 128
E_PAD = 327680
ROWS_PER_W = E_PAD // NS // CHUNK
HALF = 5056
ROWS_PER_TILE = 320
_N_STAGE = 3
NPAD_SC = 5248
CNT_W = 16
PAD_DST = NC * HALF


def _relu_body(x_ref, o_ref):
    o_ref[...] = jnp.maximum(x_ref[...], 0.0)


def _relu(x):
    return pl.pallas_call(
        _relu_body,
        out_shape=jax.ShapeDtypeStruct((N, D), jnp.float32),
    )(x)


def _sc_body(h_hbm, src_hbm, dst_hbm, iota_hbm,
             agg_out,
             src_idx, dst_idx, dst_loc, iota_idx, rows_v,
             sh_agg, sem):
    c = lax.axis_index("c")
    s = lax.axis_index("s")

    z16 = jnp.zeros((16,), jnp.float32)
    o16 = jnp.ones((16,), jnp.float32)

    @pl.loop(0, CHUNK)
    def _fill_rows(r):
        for j in range(D // 16):
            rows_v[r, pl.ds(j * 16, 16)] = z16

    r0 = s * ROWS_PER_TILE
    for k in range(_N_STAGE):
        pltpu.sync_copy(iota_hbm.at[pl.ds(r0 + k * CHUNK, CHUNK)], iota_idx)
        pltpu.sync_copy(rows_v, sh_agg.at[iota_idx])

    plsc.subcore_barrier()

    base_node = c * HALF
    e0 = s * ROWS_PER_W * CHUNK

    @pl.loop(0, ROWS_PER_W)
    def _edges(i):
        base = e0 + i * CHUNK
        pltpu.sync_copy(src_hbm.at[pl.ds(base, CHUNK)], src_idx)
        pltpu.sync_copy(dst_hbm.at[pl.ds(base, CHUNK)], dst_idx)
        gat = pltpu.async_copy(h_hbm.at[src_idx], rows_v, sem)
        for j in range(CHUNK // 16):
            t = dst_idx[pl.ds(j * 16, 16)] - base_node
            ok = (t >= 0) & (t < HALF)
            dst_loc[pl.ds(j * 16, 16)] = jnp.where(ok, t, HALF)
        gat.wait()
        pltpu.sync_copy(rows_v, sh_agg.at[dst_loc], add=True)

    plsc.subcore_barrier()

    for k in range(_N_STAGE):
        pltpu.sync_copy(iota_hbm.at[pl.ds(r0 + k * CHUNK, CHUNK)], iota_idx)
        pltpu.async_copy(sh_agg.at[iota_idx], rows_v, sem).wait()
        pltpu.sync_copy(rows_v, agg_out.at[c, pl.ds(r0 + k * CHUNK, CHUNK)])


def _sc_aggregate(h, src, dst, iota):
    mesh = plsc.VectorSubcoreMesh(core_axis_name="c", subcore_axis_name="s")
    k = pl.kernel(
        _sc_body,
        out_type=jax.ShapeDtypeStruct((NC, NPAD_SC, D), jnp.float32),
        mesh=mesh,
        scratch_types=[
            pltpu.VMEM((CHUNK,), jnp.int32),
            pltpu.VMEM((CHUNK,), jnp.int32),
            pltpu.VMEM((CHUNK,), jnp.int32),
            pltpu.VMEM((CHUNK,), jnp.int32),
            pltpu.VMEM((CHUNK, D), jnp.float32),
            pltpu.VMEM_SHARED((NPAD_SC, D), jnp.float32),
            pltpu.SemaphoreType.DMA,
        ],
    )
    return k(h, src, dst, iota)


def _cnt_body(dst_hbm, iota_hbm,
              cnt_out,
              dst_idx, dst_loc, iota_idx, z_v, o_v,
              sh_cnt, sem):
    c = lax.axis_index("c")
    s = lax.axis_index("s")

    z16 = jnp.zeros((16,), jnp.float32)
    o16 = jnp.ones((16,), jnp.float32)

    @pl.loop(0, CHUNK)
    def _fill(r):
        for j in range(D // 16):
            z_v[r, pl.ds(j * 16, 16)] = z16
            o_v[r, pl.ds(j * 16, 16)] = o16

    r0 = s * ROWS_PER_TILE
    for k in range(_N_STAGE):
        pltpu.sync_copy(iota_hbm.at[pl.ds(r0 + k * CHUNK, CHUNK)], iota_idx)
        pltpu.sync_copy(z_v, sh_cnt.at[iota_idx])

    plsc.subcore_barrier()

    base_node = c * HALF
    e0 = s * ROWS_PER_W * CHUNK

    @pl.loop(0, ROWS_PER_W)
    def _edges(i):
        base = e0 + i * CHUNK
        pltpu.sync_copy(dst_hbm.at[pl.ds(base, CHUNK)], dst_idx)
        for j in range(CHUNK // 16):
            t = dst_idx[pl.ds(j * 16, 16)] - base_node
            ok = (t >= 0) & (t < HALF)
            dst_loc[pl.ds(j * 16, 16)] = jnp.where(ok, t, HALF)
        pltpu.sync_copy(o_v, sh_cnt.at[dst_loc], add=True)

    plsc.subcore_barrier()

    for k in range(_N_STAGE):
        pltpu.sync_copy(iota_hbm.at[pl.ds(r0 + k * CHUNK, CHUNK)], iota_idx)
        pltpu.async_copy(sh_cnt.at[iota_idx], z_v, sem).wait()
        pltpu.sync_copy(z_v, cnt_out.at[c, pl.ds(r0 + k * CHUNK, CHUNK)])


def _sc_counts(dst, iota):
    mesh = plsc.VectorSubcoreMesh(core_axis_name="c", subcore_axis_name="s")
    k = pl.kernel(
        _cnt_body,
        out_type=jax.ShapeDtypeStruct((NC, NPAD_SC, D), jnp.float32),
        mesh=mesh,
        scratch_types=[
            pltpu.VMEM((CHUNK,), jnp.int32),
            pltpu.VMEM((CHUNK,), jnp.int32),
            pltpu.VMEM((CHUNK,), jnp.int32),
            pltpu.VMEM((CHUNK, D), jnp.float32),
            pltpu.VMEM((CHUNK, D), jnp.float32),
            pltpu.VMEM_SHARED((NPAD_SC, D), jnp.float32),
            pltpu.SemaphoreType.DMA,
        ],
    )
    return k(dst, iota)


def _combine_body(h_ref, agg_ref, cnt_ref, wl_ref, bl_ref, wr_ref,
                  g_ref, b_ref, o_ref):
    agg = jnp.concatenate(
        [agg_ref[0, :HALF, :], agg_ref[1, :N - HALF, :]], axis=0)
    cnt = jnp.concatenate(
        [cnt_ref[0, :HALF, 0:1], cnt_ref[1, :N - HALF, 0:1]], axis=0)
    mean_agg = agg / jnp.maximum(cnt, 1.0)
    h = h_ref[...]
    out = (
        lax.dot_general(mean_agg, wl_ref[...], (((1,), (1,)), ((), ())),
                        preferred_element_type=jnp.float32)
        + bl_ref[...]
        + lax.dot_general(h, wr_ref[...], (((1,), (1,)), ((), ())),
                          preferred_element_type=jnp.float32)
    )
    mu = jnp.mean(out, axis=0, keepdims=True)
    ctr = out - mu
    var = jnp.mean(ctr * ctr, axis=0, keepdims=True)
    o_ref[...] = ctr * (g_ref[...] * lax.rsqrt(var + 1e-5)) + b_ref[...]


def _combine(h, agg, cnt, Wl, bl, Wr, gamma, beta):
    return pl.pallas_call(
        _combine_body,
        out_shape=jax.ShapeDtypeStruct((N, D), jnp.float32),
    )(h, agg, cnt, Wl, bl.reshape(1, D), Wr,
      gamma.reshape(1, D), beta.reshape(1, D))


def kernel(x, edge_index, edge_attr, Wl, bl, Wr, gamma, beta):
    del edge_attr
    h = _relu(x)

    pad = E_PAD - E
    src = jnp.concatenate([edge_index[0], jnp.zeros((pad,), jnp.int32)])
    dst = jnp.concatenate([edge_index[1], jnp.full((pad,), PAD_DST, jnp.int32)])
    iota = jnp.arange(NPAD_SC, dtype=jnp.int32)
    agg = _sc_aggregate(h, src, dst, iota)
    cnt = _sc_counts(dst, iota)
    return _combine(h, agg, cnt, Wl, bl, Wr, gamma, beta)


# speedup vs baseline: 2.0364x; 1.0616x over previous
import jax
import jax.numpy as jnp
from jax import lax
from jax.experimental import pallas as pl
from jax.experimental.pallas import tpu as pltpu
from jax.experimental.pallas import tpu_sc as plsc

N = 10000
D = 128
E = 320000
NC = 2
NS = 16

CHUNK = 128
E_PAD = 327680
ROWS_PER_W = E_PAD // NS // CHUNK
HALF = 5056
ROWS_PER_TILE = 320
_N_STAGE = 3
NPAD_SC = 5248
CNT_W = 16
PAD_DST = NC * HALF
_KB = 4


def _relu_body(x_ref, o_ref):
    o_ref[...] = jnp.maximum(x_ref[...], 0.0)


def _relu(x):
    return pl.pallas_call(
        _relu_body,
        out_shape=jax.ShapeDtypeStruct((N, D), jnp.float32),
    )(x)


def _sc_body(h_hbm, src_hbm, dst_hbm, iota_hbm,
             agg_out,
             si0, si1, si2, si3,
             di0, di1, di2, di3,
             dl0, dl1, dl2, dl3,
             rv0, rv1, rv2, rv3,
             iota_idx,
             sh_agg, sem):
    c = lax.axis_index("c")
    s = lax.axis_index("s")
    src_idx = [si0, si1, si2, si3]
    dst_idx = [di0, di1, di2, di3]
    dst_loc = [dl0, dl1, dl2, dl3]
    rows_v = [rv0, rv1, rv2, rv3]

    z16 = jnp.zeros((16,), jnp.float32)

    @pl.loop(0, CHUNK)
    def _fill_rows(r):
        for j in range(D // 16):
            rv0[r, pl.ds(j * 16, 16)] = z16

    r0 = s * ROWS_PER_TILE
    for k in range(_N_STAGE):
        pltpu.sync_copy(iota_hbm.at[pl.ds(r0 + k * CHUNK, CHUNK)], iota_idx)
        pltpu.sync_copy(rv0, sh_agg.at[iota_idx])

    plsc.subcore_barrier()

    base_node = c * HALF
    e0 = s * ROWS_PER_W * CHUNK

    @pl.loop(0, ROWS_PER_W // _KB)
    def _edges(g):
        base = e0 + g * (_KB * CHUNK)
        gats = []
        for b in range(_KB):
            pltpu.sync_copy(src_hbm.at[pl.ds(base + b * CHUNK, CHUNK)],
                            src_idx[b])
            pltpu.sync_copy(dst_hbm.at[pl.ds(base + b * CHUNK, CHUNK)],
                            dst_idx[b])
            gats.append(pltpu.async_copy(h_hbm.at[src_idx[b]], rows_v[b], sem))
        for b in range(_KB):
            for j in range(CHUNK // 16):
                t = dst_idx[b][pl.ds(j * 16, 16)] - base_node
                ok = (t >= 0) & (t < HALF)
                dst_loc[b][pl.ds(j * 16, 16)] = jnp.where(ok, t, HALF)
        for gat in gats:
            gat.wait()
        for b in range(_KB):
            pltpu.sync_copy(rows_v[b], sh_agg.at[dst_loc[b]], add=True)

    plsc.subcore_barrier()

    for k in range(_N_STAGE):
        pltpu.sync_copy(iota_hbm.at[pl.ds(r0 + k * CHUNK, CHUNK)], iota_idx)
        pltpu.async_copy(sh_agg.at[iota_idx], rv0, sem).wait()
        pltpu.sync_copy(rv0, agg_out.at[c, pl.ds(r0 + k * CHUNK, CHUNK)])


def _sc_aggregate(h, src, dst, iota):
    mesh = plsc.VectorSubcoreMesh(core_axis_name="c", subcore_axis_name="s")
    k = pl.kernel(
        _sc_body,
        out_type=jax.ShapeDtypeStruct((NC, NPAD_SC, D), jnp.float32),
        mesh=mesh,
        scratch_types=(
            [pltpu.VMEM((CHUNK,), jnp.int32)] * (3 * _KB)
            + [pltpu.VMEM((CHUNK, D), jnp.float32)] * _KB
            + [pltpu.VMEM((CHUNK,), jnp.int32),
               pltpu.VMEM_SHARED((NPAD_SC, D), jnp.float32),
               pltpu.SemaphoreType.DMA]
        ),
    )
    return k(h, src, dst, iota)


def _cnt_body(dst_hbm, iota_hbm,
              cnt_out,
              dst_idx, dst_loc, iota_idx, z_v, o_v,
              sh_cnt, sem):
    c = lax.axis_index("c")
    s = lax.axis_index("s")

    z16 = jnp.zeros((16,), jnp.float32)
    o16 = jnp.ones((16,), jnp.float32)

    @pl.loop(0, CHUNK)
    def _fill(r):
        for j in range(D // 16):
            z_v[r, pl.ds(j * 16, 16)] = z16
            o_v[r, pl.ds(j * 16, 16)] = o16

    r0 = s * ROWS_PER_TILE
    for k in range(_N_STAGE):
        pltpu.sync_copy(iota_hbm.at[pl.ds(r0 + k * CHUNK, CHUNK)], iota_idx)
        pltpu.sync_copy(z_v, sh_cnt.at[iota_idx])

    plsc.subcore_barrier()

    base_node = c * HALF
    e0 = s * ROWS_PER_W * CHUNK

    @pl.loop(0, ROWS_PER_W)
    def _edges(i):
        base = e0 + i * CHUNK
        pltpu.sync_copy(dst_hbm.at[pl.ds(base, CHUNK)], dst_idx)
        for j in range(CHUNK // 16):
            t = dst_idx[pl.ds(j * 16, 16)] - base_node
            ok = (t >= 0) & (t < HALF)
            dst_loc[pl.ds(j * 16, 16)] = jnp.where(ok, t, HALF)
        pltpu.sync_copy(o_v, sh_cnt.at[dst_loc], add=True)

    plsc.subcore_barrier()

    for k in range(_N_STAGE):
        pltpu.sync_copy(iota_hbm.at[pl.ds(r0 + k * CHUNK, CHUNK)], iota_idx)
        pltpu.async_copy(sh_cnt.at[iota_idx], z_v, sem).wait()
        pltpu.sync_copy(z_v, cnt_out.at[c, pl.ds(r0 + k * CHUNK, CHUNK)])


def _sc_counts(dst, iota):
    mesh = plsc.VectorSubcoreMesh(core_axis_name="c", subcore_axis_name="s")
    k = pl.kernel(
        _cnt_body,
        out_type=jax.ShapeDtypeStruct((NC, NPAD_SC, D), jnp.float32),
        mesh=mesh,
        scratch_types=[
            pltpu.VMEM((CHUNK,), jnp.int32),
            pltpu.VMEM((CHUNK,), jnp.int32),
            pltpu.VMEM((CHUNK,), jnp.int32),
            pltpu.VMEM((CHUNK, D), jnp.float32),
            pltpu.VMEM((CHUNK, D), jnp.float32),
            pltpu.VMEM_SHARED((NPAD_SC, D), jnp.float32),
            pltpu.SemaphoreType.DMA,
        ],
    )
    return k(dst, iota)


def _combine_body(h_ref, agg_ref, cnt_ref, wl_ref, bl_ref, wr_ref,
                  g_ref, b_ref, o_ref):
    agg = jnp.concatenate(
        [agg_ref[0, :HALF, :], agg_ref[1, :N - HALF, :]], axis=0)
    cnt = jnp.concatenate(
        [cnt_ref[0, :HALF, 0:1], cnt_ref[1, :N - HALF, 0:1]], axis=0)
    mean_agg = agg / jnp.maximum(cnt, 1.0)
    h = h_ref[...]
    out = (
        lax.dot_general(mean_agg, wl_ref[...], (((1,), (1,)), ((), ())),
                        preferred_element_type=jnp.float32)
        + bl_ref[...]
        + lax.dot_general(h, wr_ref[...], (((1,), (1,)), ((), ())),
                          preferred_element_type=jnp.float32)
    )
    mu = jnp.mean(out, axis=0, keepdims=True)
    ctr = out - mu
    var = jnp.mean(ctr * ctr, axis=0, keepdims=True)
    o_ref[...] = ctr * (g_ref[...] * lax.rsqrt(var + 1e-5)) + b_ref[...]


def _combine(h, agg, cnt, Wl, bl, Wr, gamma, beta):
    return pl.pallas_call(
        _combine_body,
        out_shape=jax.ShapeDtypeStruct((N, D), jnp.float32),
    )(h, agg, cnt, Wl, bl.reshape(1, D), Wr,
      gamma.reshape(1, D), beta.reshape(1, D))


def kernel(x, edge_index, edge_attr, Wl, bl, Wr, gamma, beta):
    del edge_attr
    h = _relu(x)

    pad = E_PAD - E
    src = jnp.concatenate([edge_index[0], jnp.zeros((pad,), jnp.int32)])
    dst = jnp.concatenate([edge_index[1], jnp.full((pad,), PAD_DST, jnp.int32)])
    iota = jnp.arange(NPAD_SC, dtype=jnp.int32)
    agg = _sc_aggregate(h, src, dst, iota)
    cnt = _sc_counts(dst, iota)
    return _combine(h, agg, cnt, Wl, bl, Wr, gamma, beta)
